# baseline (speedup 1.0000x reference)
import functools

import jax
import jax.numpy as jnp
from jax import lax
from jax.experimental import pallas as pl
from jax.experimental.pallas import tpu as pltpu
from jax.experimental.pallas import tpu_sc as plsc

LANES = 16
CHUNK = 64
FDIM = 128
N_SUB = 16
NBUF = 3


def _edge_agg_body(nchunks_ps, nvalid_ps, n_nodes,
                   x_hbm, srcf_hbm, dstf_hbm, acc_hbm, dego_hbm, degi_hbm,
                   r0, r1, r2, sring_v, dring_v, deglo_v, degli_v,
                   acc_sh, g0, g1, g2, i0, i1, i2):
    rows_v = [r0, r1, r2]
    semg = [g0, g1, g2]
    semi = [i0, i1, i2]
    cid = lax.axis_index("c")
    sid = lax.axis_index("s")
    wid = cid * N_SUB + sid
    nfly = NBUF - 1
    zrows = CHUNK
    nzchunks = n_nodes // zrows
    zk = (nzchunks + N_SUB - 1) // N_SUB
    base = wid * nchunks_ps * CHUNK

    for k in range(NBUF):
        pltpu.async_copy(srcf_hbm.at[pl.ds(base + k * CHUNK, CHUNK)],
                         sring_v.at[k], semi[k])
        pltpu.async_copy(dstf_hbm.at[pl.ds(base + k * CHUNK, CHUNK)],
                         dring_v.at[k], semi[k])

    def _zb(i, _):
        r0[i // (FDIM // LANES), pl.ds((i % (FDIM // LANES)) * LANES, LANES)] = (
            jnp.zeros((LANES,), jnp.float32))
        return 0
    lax.fori_loop(0, CHUNK * (FDIM // LANES), _zb, 0)

    def _zd(i, _):
        deglo_v[pl.ds(i * LANES, LANES)] = jnp.zeros((LANES,), jnp.float32)
        degli_v[pl.ds(i * LANES, LANES)] = jnp.zeros((LANES,), jnp.float32)
        return 0
    lax.fori_loop(0, n_nodes // LANES, _zd, 0)

    def _zacc(k, _):
        j = sid + k * N_SUB
        @pl.when(j < nzchunks)
        def _():
            pltpu.sync_copy(r0, acc_sh.at[pl.ds(j * zrows, zrows)])
        return 0
    lax.fori_loop(0, zk, _zacc, 0)

    plsc.subcore_barrier()

    ones16 = jnp.ones((LANES,), jnp.float32)
    vpc = CHUNK // LANES
    nvregs = nvalid_ps // LANES
    idx_bytes = CHUNK * 4

    for k in range(nfly):
        pltpu.make_async_copy(srcf_hbm.at[pl.ds(base, CHUNK)], sring_v.at[k],
                              semi[k]).wait()
        pltpu.make_async_copy(dstf_hbm.at[pl.ds(base, CHUNK)], dring_v.at[k],
                              semi[k]).wait()
        pltpu.async_copy(x_hbm.at[sring_v.at[k]], rows_v[k], semg[k])

    def _edge(i, _):
        j0 = NBUF * i
        for k in range(NBUF):
            j = j0 + k
            kn = (k + nfly) % NBUF
            pltpu.make_async_copy(x_hbm.at[sring_v.at[k]], rows_v[k], semg[k]).wait()
            pltpu.make_async_copy(srcf_hbm.at[pl.ds(base, CHUNK)],
                                  sring_v.at[kn], semi[kn]).wait()
            pltpu.make_async_copy(dstf_hbm.at[pl.ds(base, CHUNK)],
                                  dring_v.at[kn], semi[kn]).wait()
            pltpu.async_copy(x_hbm.at[sring_v.at[kn]], rows_v[kn], semg[kn])
            for c in range(vpc):
                @pl.when(j * vpc + c < nvregs)
                def _():
                    idxs = sring_v[k, pl.ds(c * LANES, LANES)]
                    plsc.addupdate_scatter(deglo_v, [idxs], ones16)
                    idxd = dring_v[k, pl.ds(c * LANES, LANES)]
                    plsc.addupdate_scatter(degli_v, [idxd], ones16)
            pltpu.sync_copy(rows_v[k], acc_sh.at[dring_v.at[k]], add=True)
            jf = jnp.where(j + NBUF < nchunks_ps, j + NBUF, 0)
            pltpu.async_copy(srcf_hbm.at[pl.ds(base + jf * CHUNK, CHUNK)],
                             sring_v.at[k], semi[k])
            pltpu.async_copy(dstf_hbm.at[pl.ds(base + jf * CHUNK, CHUNK)],
                             dring_v.at[k], semi[k])
        return 0
    lax.fori_loop(0, nchunks_ps // NBUF, _edge, 0)
    klast = (nchunks_ps - 1) % NBUF
    pltpu.make_async_copy(srcf_hbm.at[pl.ds(base, CHUNK)], sring_v.at[klast],
                          semi[klast]).wait()
    pltpu.make_async_copy(dstf_hbm.at[pl.ds(base, CHUNK)], dring_v.at[klast],
                          semi[klast]).wait()
    for k in range(nfly):
        pltpu.make_async_copy(x_hbm.at[sring_v.at[k]], rows_v[k], semg[k]).wait()

    pltpu.sync_copy(deglo_v, dego_hbm.at[pl.ds(wid * n_nodes, n_nodes)])
    pltpu.sync_copy(degli_v, degi_hbm.at[pl.ds(wid * n_nodes, n_nodes)])

    plsc.subcore_barrier()

    def _wacc(k, _):
        j = sid + k * N_SUB
        @pl.when(j < nzchunks)
        def _():
            off = j * zrows
            pltpu.sync_copy(acc_sh.at[pl.ds(off, zrows)], r0)
            pltpu.sync_copy(r0, acc_hbm.at[cid, pl.ds(off, zrows)])
        return 0
    lax.fori_loop(0, zk, _wacc, 0)


def _edge_aggregate(x, srcf, dstf, nchunks_ps, nvalid_ps, n_nodes):
    mesh = plsc.VectorSubcoreMesh(core_axis_name="c", subcore_axis_name="s")
    kern = pl.kernel(
        functools.partial(_edge_agg_body, nchunks_ps, nvalid_ps, n_nodes),
        out_type=(
            jax.ShapeDtypeStruct((2, n_nodes, FDIM), jnp.float32),
            jax.ShapeDtypeStruct((2 * N_SUB * n_nodes,), jnp.float32),
            jax.ShapeDtypeStruct((2 * N_SUB * n_nodes,), jnp.float32),
        ),
        mesh=mesh,
        compiler_params=pltpu.CompilerParams(use_tc_tiling_on_sc=False,
                                             needs_layout_passes=False),
        scratch_types=(
            pltpu.VMEM((CHUNK, FDIM), jnp.float32),
            pltpu.VMEM((CHUNK, FDIM), jnp.float32),
            pltpu.VMEM((CHUNK, FDIM), jnp.float32),
            pltpu.VMEM((NBUF, CHUNK), jnp.int32),
            pltpu.VMEM((NBUF, CHUNK), jnp.int32),
            pltpu.VMEM((n_nodes,), jnp.float32),
            pltpu.VMEM((n_nodes,), jnp.float32),
            pltpu.VMEM_SHARED((n_nodes + 8, FDIM), jnp.float32),
            pltpu.SemaphoreType.DMA,
            pltpu.SemaphoreType.DMA,
            pltpu.SemaphoreType.DMA,
            pltpu.SemaphoreType.DMA,
            pltpu.SemaphoreType.DMA,
            pltpu.SemaphoreType.DMA,
        ),
    )
    return kern(x, srcf, dstf)


def _combine_body(n_edges, acc_ref, dego_ref, degi_ref, w1_ref, b1_ref,
                  w2_ref, b2_ref, o_ref):
    aggx = acc_ref[0] + acc_ref[1]
    outdeg = jnp.sum(dego_ref[...], axis=0)
    indeg = jnp.sum(degi_ref[...], axis=0)
    agg1 = lax.dot_general(
        aggx, w1_ref[...], (((1,), (1,)), ((), ())),
        preferred_element_type=jnp.float32) + indeg[:, None] * b1_ref[...][None, :]
    t = jnp.tanh(agg1)
    w2s = jnp.sum(w2_ref[...], axis=0)
    total = jnp.sum(jnp.sum(t * w2s[None, :], axis=1) * outdeg)
    total = total + n_edges * jnp.sum(b2_ref[...])
    o_ref[...] = total[None, None]


def _combine(acc, dego, degi, W1, b1, W2, b2, n_edges):
    return pl.pallas_call(
        functools.partial(_combine_body, float(n_edges)),
        out_shape=jax.ShapeDtypeStruct((1, 1), jnp.float32),
    )(acc, dego, degi, W1, b1, W2, b2)


def kernel(x, edge_index, W1, b1, W2, b2):
    n_nodes = x.shape[0]
    n_edges = edge_index.shape[1]
    nvalid_ps = n_edges // (2 * N_SUB)
    nck = -(-nvalid_ps // CHUNK)
    nck += (-nck) % NBUF
    npad = nck * CHUNK - nvalid_ps
    srcf = jnp.concatenate(
        [edge_index[0].reshape(2 * N_SUB, nvalid_ps),
         jnp.zeros((2 * N_SUB, npad), jnp.int32)], axis=1).reshape(-1)
    dstf = jnp.concatenate(
        [edge_index[1].reshape(2 * N_SUB, nvalid_ps),
         jnp.full((2 * N_SUB, npad), n_nodes, jnp.int32)], axis=1).reshape(-1)

    acc, dego, degi = _edge_aggregate(x, srcf, dstf, nck, nvalid_ps, n_nodes)
    return _combine(acc, dego.reshape(2 * N_SUB, n_nodes),
                    degi.reshape(2 * N_SUB, n_nodes), W1, b1, W2, b2, n_edges)


# speedup vs baseline: 2.6045x; 2.6045x over previous
import functools

import jax
import jax.numpy as jnp
from jax import lax
from jax.experimental import pallas as pl
from jax.experimental.pallas import tpu as pltpu
from jax.experimental.pallas import tpu_sc as plsc

LANES = 16
CHUNK = 80
HALF = 64
N_SUB = 16


def _linear_body(x_ref, w_ref, b_ref, o_ref):
    h = lax.dot_general(
        x_ref[...], w_ref[...], (((1,), (1,)), ((), ())),
        preferred_element_type=jnp.float32) + b_ref[...][None, :]
    o_ref[0] = h[:, :HALF]
    o_ref[1] = h[:, HALF:]


def _linear_split(x, W, b):
    n, _ = x.shape
    return pl.pallas_call(
        _linear_body,
        out_shape=jax.ShapeDtypeStruct((2, n, HALF), jnp.float32),
    )(x, W, b)


def _edge_agg_body(nchunks_ps, n_nodes,
                   h1_hbm, srcm_hbm, dstm_hbm, acc_hbm, deg_hbm,
                   r0, r1, r2, r3, r4, src_v, dst_v, zbuf_v, degl_v,
                   acc_sh, s0, s1, s2, s3, s4, t0, t1, t2, t3, t4):
    rows_v = [r0, r1, r2, r3, r4]
    semg = [s0, s1, s2, s3, s4]
    sems = [t0, t1, t2, t3, t4]
    cid = lax.axis_index("c")
    sid = lax.axis_index("s")
    zrows = zbuf_v.shape[0]
    nzchunks = n_nodes // zrows
    zk = (nzchunks + N_SUB - 1) // N_SUB
    nbuf = len(rows_v)

    ds = pltpu.async_copy(srcm_hbm.at[sid], src_v, semg[0])
    dd = pltpu.async_copy(dstm_hbm.at[sid], dst_v, semg[1])

    def _z2(i, _):
        r = i // (HALF // LANES)
        c = i % (HALF // LANES)
        zbuf_v[r, pl.ds(c * LANES, LANES)] = jnp.zeros((LANES,), jnp.float32)
        return 0
    lax.fori_loop(0, zrows * (HALF // LANES), _z2, 0)

    def _zd(i, _):
        degl_v[pl.ds(i * LANES, LANES)] = jnp.zeros((LANES,), jnp.float32)
        return 0
    lax.fori_loop(0, n_nodes // LANES, _zd, 0)

    ds.wait()
    dd.wait()

    def _zacc(k, _):
        j = sid + k * N_SUB
        @pl.when(j < nzchunks)
        def _():
            pltpu.sync_copy(zbuf_v, acc_sh.at[pl.ds(j * zrows, zrows)])
        return 0
    lax.fori_loop(0, zk, _zacc, 0)

    plsc.subcore_barrier()

    table = h1_hbm.at[cid]
    ones16 = jnp.ones((LANES,), jnp.float32)
    vpc = CHUNK // LANES
    nfly = nbuf - 1
    for k in range(nfly):
        pltpu.async_copy(table.at[src_v.at[k]], rows_v[k], semg[k])

    def _edge(i, _):
        j0 = nbuf * i
        for k in range(nbuf):
            j = j0 + k
            kr = (k + nfly) % nbuf
            pltpu.make_async_copy(table.at[src_v.at[j]], rows_v[k], semg[k]).wait()
            pltpu.async_copy(rows_v[k], acc_sh.at[dst_v.at[j]], sems[k], add=True)
            if k == 0:
                @pl.when(i > 0)
                def _():
                    pltpu.make_async_copy(rows_v[kr], acc_sh.at[dst_v.at[0]],
                                          sems[kr]).wait()
            else:
                pltpu.make_async_copy(rows_v[kr], acc_sh.at[dst_v.at[0]],
                                      sems[kr]).wait()
            jn = jnp.where(j + nfly < nchunks_ps, j + nfly, 0)
            pltpu.async_copy(table.at[src_v.at[jn]], rows_v[kr], semg[kr])
            for c in range(vpc):
                idx = src_v[j, pl.ds(c * LANES, LANES)]
                plsc.addupdate_scatter(degl_v, [idx], ones16)
        return 0
    lax.fori_loop(0, nchunks_ps // nbuf, _edge, 0)
    pltpu.make_async_copy(rows_v[(nchunks_ps - 1) % nbuf], acc_sh.at[dst_v.at[0]],
                          sems[(nchunks_ps - 1) % nbuf]).wait()
    for k in range(nfly):
        pltpu.make_async_copy(table.at[src_v.at[0]], rows_v[k], semg[k]).wait()

    pltpu.sync_copy(degl_v,
                    deg_hbm.at[pl.ds((cid * N_SUB + sid) * n_nodes, n_nodes)])

    plsc.subcore_barrier()

    def _wacc(k, _):
        j = sid + k * N_SUB
        @pl.when(j < nzchunks)
        def _():
            off = j * zrows
            pltpu.sync_copy(acc_sh.at[pl.ds(off, zrows)], zbuf_v)
            pltpu.sync_copy(zbuf_v, acc_hbm.at[cid, pl.ds(off, zrows)])
        return 0
    lax.fori_loop(0, zk, _wacc, 0)


def _edge_aggregate(h1s, srcm, dstm, n_nodes):
    nchunks_ps = srcm.shape[1]
    mesh = plsc.VectorSubcoreMesh(core_axis_name="c", subcore_axis_name="s")
    kern = pl.kernel(
        functools.partial(_edge_agg_body, nchunks_ps, n_nodes),
        out_type=(
            jax.ShapeDtypeStruct((2, n_nodes, HALF), jnp.float32),
            jax.ShapeDtypeStruct((2 * N_SUB * n_nodes,), jnp.float32),
        ),
        mesh=mesh,
        compiler_params=pltpu.CompilerParams(use_tc_tiling_on_sc=False,
                                             needs_layout_passes=False),
        scratch_types=(
            pltpu.VMEM((CHUNK, HALF), jnp.float32),
            pltpu.VMEM((CHUNK, HALF), jnp.float32),
            pltpu.VMEM((CHUNK, HALF), jnp.float32),
            pltpu.VMEM((CHUNK, HALF), jnp.float32),
            pltpu.VMEM((CHUNK, HALF), jnp.float32),
            pltpu.VMEM((nchunks_ps, CHUNK), jnp.int32),
            pltpu.VMEM((nchunks_ps, CHUNK), jnp.int32),
            pltpu.VMEM((200, HALF), jnp.float32),
            pltpu.VMEM((n_nodes,), jnp.float32),
            pltpu.VMEM_SHARED((n_nodes, HALF), jnp.float32),
            pltpu.SemaphoreType.DMA,
            pltpu.SemaphoreType.DMA,
            pltpu.SemaphoreType.DMA,
            pltpu.SemaphoreType.DMA,
            pltpu.SemaphoreType.DMA,
            pltpu.SemaphoreType.DMA,
            pltpu.SemaphoreType.DMA,
            pltpu.SemaphoreType.DMA,
            pltpu.SemaphoreType.DMA,
            pltpu.SemaphoreType.DMA,
        ),
    )
    return kern(h1s, srcm, dstm)


def _combine_body(n_edges, acc_ref, deg_ref, w2_ref, b2_ref, o_ref):
    w2s = jnp.sum(w2_ref[...], axis=0)
    deg = 0.5 * jnp.sum(deg_ref[...], axis=0)
    ta = jnp.tanh(acc_ref[0])
    tb = jnp.tanh(acc_ref[1])
    row = jnp.sum(ta * w2s[None, :HALF], axis=1) + jnp.sum(tb * w2s[None, HALF:], axis=1)
    total = jnp.sum(row * deg) + n_edges * jnp.sum(b2_ref[...])
    o_ref[...] = total[None, None]


def _combine(acc, deg, W2, b2, n_edges):
    return pl.pallas_call(
        functools.partial(_combine_body, float(n_edges)),
        out_shape=jax.ShapeDtypeStruct((1, 1), jnp.float32),
    )(acc, deg, W2, b2)


def kernel(x, edge_index, W1, b1, W2, b2):
    n_nodes = x.shape[0]
    n_edges = edge_index.shape[1]
    src = edge_index[0].reshape(N_SUB, n_edges // (N_SUB * CHUNK), CHUNK)
    dst = edge_index[1].reshape(N_SUB, n_edges // (N_SUB * CHUNK), CHUNK)

    h1s = _linear_split(x, W1, b1)
    acc, deg = _edge_aggregate(h1s, src, dst, n_nodes)
    return _combine(acc, deg.reshape(2 * N_SUB, n_nodes), W2, b2, n_edges)


# speedup vs baseline: 2.6261x; 1.0083x over previous
import functools

import jax
import jax.numpy as jnp
from jax import lax
from jax.experimental import pallas as pl
from jax.experimental.pallas import tpu as pltpu
from jax.experimental.pallas import tpu_sc as plsc

LANES = 16
CHUNK = 80
HALF = 64
N_SUB = 16


def _linear_body(x_ref, w_ref, b_ref, o_ref):
    h = lax.dot_general(
        x_ref[...], w_ref[...], (((1,), (1,)), ((), ())),
        preferred_element_type=jnp.float32) + b_ref[...][None, :]
    o_ref[0] = h[:, :HALF]
    o_ref[1] = h[:, HALF:]


def _linear_split(x, W, b):
    n, _ = x.shape
    return pl.pallas_call(
        _linear_body,
        out_shape=jax.ShapeDtypeStruct((2, n, HALF), jnp.float32),
    )(x, W, b)


def _edge_agg_body(nchunks_ps, n_nodes,
                   h1_hbm, srcm_hbm, dstm_hbm, acc_hbm, deg_hbm,
                   r0, r1, r2, r3, r4, src_v, dst_v, zbuf_v, degl_v,
                   acc_sh, s0, s1, s2, s3, s4):
    rows_v = [r0, r1, r2, r3, r4]
    semg = [s0, s1, s2, s3, s4]
    cid = lax.axis_index("c")
    sid = lax.axis_index("s")
    zrows = zbuf_v.shape[0]
    nzchunks = n_nodes // zrows
    zk = (nzchunks + N_SUB - 1) // N_SUB
    nbuf = len(rows_v)

    ds = pltpu.async_copy(srcm_hbm.at[sid], src_v, semg[0])
    dd = pltpu.async_copy(dstm_hbm.at[sid], dst_v, semg[1])

    def _z2(i, _):
        r = i // (HALF // LANES)
        c = i % (HALF // LANES)
        zbuf_v[r, pl.ds(c * LANES, LANES)] = jnp.zeros((LANES,), jnp.float32)
        return 0
    lax.fori_loop(0, zrows * (HALF // LANES), _z2, 0)

    def _zd(i, _):
        degl_v[pl.ds(i * LANES, LANES)] = jnp.zeros((LANES,), jnp.float32)
        return 0
    lax.fori_loop(0, n_nodes // LANES, _zd, 0)

    ds.wait()
    dd.wait()

    def _zacc(k, _):
        j = sid + k * N_SUB
        @pl.when(j < nzchunks)
        def _():
            pltpu.sync_copy(zbuf_v, acc_sh.at[pl.ds(j * zrows, zrows)])
        return 0
    lax.fori_loop(0, zk, _zacc, 0)

    plsc.subcore_barrier()

    table = h1_hbm.at[cid]
    ones16 = jnp.ones((LANES,), jnp.float32)
    vpc = CHUNK // LANES
    nfly = nbuf - 1
    for k in range(nfly):
        pltpu.async_copy(table.at[src_v.at[k]], rows_v[k], semg[k])

    def _edge(i, _):
        j0 = nbuf * i
        for k in range(nbuf):
            j = j0 + k
            kr = (k + nfly) % nbuf
            pltpu.make_async_copy(table.at[src_v.at[j]], rows_v[k], semg[k]).wait()
            jn = jnp.where(j + nfly < nchunks_ps, j + nfly, 0)
            pltpu.async_copy(table.at[src_v.at[jn]], rows_v[kr], semg[kr])
            for c in range(vpc):
                idx = src_v[j, pl.ds(c * LANES, LANES)]
                plsc.addupdate_scatter(degl_v, [idx], ones16)
            pltpu.sync_copy(rows_v[k], acc_sh.at[dst_v.at[j]], add=True)
        return 0
    lax.fori_loop(0, nchunks_ps // nbuf, _edge, 0)
    for k in range(nfly):
        pltpu.make_async_copy(table.at[src_v.at[0]], rows_v[k], semg[k]).wait()

    pltpu.sync_copy(degl_v,
                    deg_hbm.at[pl.ds((cid * N_SUB + sid) * n_nodes, n_nodes)])

    plsc.subcore_barrier()

    def _wacc(k, _):
        j = sid + k * N_SUB
        @pl.when(j < nzchunks)
        def _():
            off = j * zrows
            pltpu.sync_copy(acc_sh.at[pl.ds(off, zrows)], zbuf_v)
            pltpu.sync_copy(zbuf_v, acc_hbm.at[cid, pl.ds(off, zrows)])
        return 0
    lax.fori_loop(0, zk, _wacc, 0)


def _edge_aggregate(h1s, srcm, dstm, n_nodes):
    nchunks_ps = srcm.shape[1]
    mesh = plsc.VectorSubcoreMesh(core_axis_name="c", subcore_axis_name="s")
    kern = pl.kernel(
        functools.partial(_edge_agg_body, nchunks_ps, n_nodes),
        out_type=(
            jax.ShapeDtypeStruct((2, n_nodes, HALF), jnp.float32),
            jax.ShapeDtypeStruct((2 * N_SUB * n_nodes,), jnp.float32),
        ),
        mesh=mesh,
        compiler_params=pltpu.CompilerParams(use_tc_tiling_on_sc=False,
                                             needs_layout_passes=False),
        scratch_types=(
            pltpu.VMEM((CHUNK, HALF), jnp.float32),
            pltpu.VMEM((CHUNK, HALF), jnp.float32),
            pltpu.VMEM((CHUNK, HALF), jnp.float32),
            pltpu.VMEM((CHUNK, HALF), jnp.float32),
            pltpu.VMEM((CHUNK, HALF), jnp.float32),
            pltpu.VMEM((nchunks_ps, CHUNK), jnp.int32),
            pltpu.VMEM((nchunks_ps, CHUNK), jnp.int32),
            pltpu.VMEM((200, HALF), jnp.float32),
            pltpu.VMEM((n_nodes,), jnp.float32),
            pltpu.VMEM_SHARED((n_nodes, HALF), jnp.float32),
            pltpu.SemaphoreType.DMA,
            pltpu.SemaphoreType.DMA,
            pltpu.SemaphoreType.DMA,
            pltpu.SemaphoreType.DMA,
            pltpu.SemaphoreType.DMA,
        ),
    )
    return kern(h1s, srcm, dstm)


def _combine_body(n_edges, acc_ref, deg_ref, w2_ref, b2_ref, o_ref):
    w2s = jnp.sum(w2_ref[...], axis=0)
    deg = 0.5 * jnp.sum(deg_ref[...], axis=0)
    ta = jnp.tanh(acc_ref[0])
    tb = jnp.tanh(acc_ref[1])
    row = jnp.sum(ta * w2s[None, :HALF], axis=1) + jnp.sum(tb * w2s[None, HALF:], axis=1)
    total = jnp.sum(row * deg) + n_edges * jnp.sum(b2_ref[...])
    o_ref[...] = total[None, None]


def _combine(acc, deg, W2, b2, n_edges):
    return pl.pallas_call(
        functools.partial(_combine_body, float(n_edges)),
        out_shape=jax.ShapeDtypeStruct((1, 1), jnp.float32),
    )(acc, deg, W2, b2)


def kernel(x, edge_index, W1, b1, W2, b2):
    n_nodes = x.shape[0]
    n_edges = edge_index.shape[1]
    src = edge_index[0].reshape(N_SUB, n_edges // (N_SUB * CHUNK), CHUNK)
    dst = edge_index[1].reshape(N_SUB, n_edges // (N_SUB * CHUNK), CHUNK)

    h1s = _linear_split(x, W1, b1)
    acc, deg = _edge_aggregate(h1s, src, dst, n_nodes)
    return _combine(acc, deg.reshape(2 * N_SUB, n_nodes), W2, b2, n_edges)
